# baseline (speedup 1.0000x reference)
import functools

import jax
import jax.numpy as jnp
from jax import lax
from jax.experimental import pallas as pl
from jax.experimental.pallas import tpu as pltpu
from jax.experimental.pallas import tpu_sc as plsc

N = 10000
E = 320000
D = 128
H = 128
C = 10
G = 64

NC = 2
NS = 16
NW = NC * NS

K = 128
NB = 81
SBB = 9
EPT = NB * K
EP = NW * EPT
NSBT = NW * (NB // SBB)
FAST = 1
NSB_F = 10
NSB_S = 8
NPAD = 10240
OROWS = 10112
ORT = OROWS // NS
DPT = NPAD // NS

_F32 = jnp.float32


def _sc_gat_body(h_hbm, asrc_hbm, adst_hbm, src_hbm, dst_hbm,
                 out_hbm, den_hbm,
                 src_v, dst_v, asb_v, adb_v, w_v, rows_v,
                 out_sh, den_sh, asrc_sh, adst_sh,
                 sem_is, sem_id, sem_as, sem_ad, sem_g, sem_so, sem_sw):
    cid = lax.axis_index("c")
    tid = lax.axis_index("s")
    wid = tid * NC + cid
    ebase = wid * EPT
    base = tid * ORT
    dbase = tid * DPT

    z16 = jnp.zeros((16,), _F32)

    def zrow(r, c2):
        for c in range(8):
            rows_v[0, r, pl.ds(c * 16, 16)] = z16
        return c2

    lax.fori_loop(0, K, zrow, 0)
    for c in range(8):
        w_v[0, pl.ds(c * 16, 16)] = z16

    @pl.when(tid == 0)
    def _():
        pltpu.sync_copy(asrc_hbm, asrc_sh)
        pltpu.sync_copy(adst_hbm, adst_sh)

    nfull = ORT // K
    for jj in range(nfull):
        pltpu.sync_copy(rows_v.at[0], out_sh.at[pl.ds(base + jj * K, K)])
    rem = ORT - nfull * K
    if rem:
        pltpu.sync_copy(rows_v.at[0, pl.ds(0, rem)],
                        out_sh.at[pl.ds(base + nfull * K, rem)])
    for jj in range(DPT // K):
        pltpu.sync_copy(w_v.at[0], den_sh.at[pl.ds(dbase + jj * K, K)])
    plsc.subcore_barrier()

    def _issue_scalar_gathers(b, p):
        cp_as = pltpu.async_copy(asrc_sh.at[src_v.at[b]], asb_v.at[p],
                                 sem_as.at[p])
        cp_ad = pltpu.async_copy(adst_sh.at[dst_v.at[b]], adb_v.at[p],
                                 sem_ad.at[p])
        return cp_as, cp_ad

    is_fast = cid == FAST
    sb_base = jnp.where(is_fast, tid * NSB_F, NS * NSB_F + tid * NSB_S)
    nsb = jnp.where(is_fast, NSB_F, NSB_S)

    def superbatch(sb, c1):
        sb0 = sb_base + sb
        pltpu.sync_copy(src_hbm.at[sb0], src_v)
        pltpu.sync_copy(dst_hbm.at[sb0], dst_v)

        sc_descs = [None, None]
        g_descs = [None, None]
        a_descs = [None, None]
        a_descs[0] = _issue_scalar_gathers(0, 0)
        g_descs[0] = pltpu.async_copy(h_hbm.at[src_v.at[0]], rows_v.at[0],
                                      sem_g.at[0])
        for b in range(SBB):
            p = b & 1
            q = 1 - p
            if b >= 1:
                for d in sc_descs[q]:
                    d.wait()
            if b + 1 < SBB:
                a_descs[q] = _issue_scalar_gathers(b + 1, q)
            for d in a_descs[p]:
                d.wait()

            def wcomp(g, c3, _p=p):
                sl = pl.ds(g * 16, 16)
                e = asb_v[_p, sl] + adb_v[_p, sl]
                e = jnp.where(e > 0, e, 0.2 * e)
                w_v[_p, sl] = jnp.exp(e)
                return c3

            lax.fori_loop(0, K // 16, wcomp, 0)

            if b + 1 < SBB:
                g_descs[q] = pltpu.async_copy(h_hbm.at[src_v.at[b + 1]],
                                              rows_v.at[q], sem_g.at[q])
            g_descs[p].wait()

            def scale(g, c3, _p=p):
                wv16 = w_v[_p, pl.ds(g * 16, 16)]
                for k in range(16):
                    wsc = wv16[k]
                    r = g * 16 + k
                    for c in range(8):
                        slc = pl.ds(c * 16, 16)
                        rows_v[_p, r, slc] = rows_v[_p, r, slc] * wsc
                return c3

            lax.fori_loop(0, K // 16, scale, 0)
            sc_descs[p] = (
                pltpu.async_copy(rows_v.at[p], out_sh.at[dst_v.at[b]],
                                 sem_so.at[p], add=True),
                pltpu.async_copy(w_v.at[p], den_sh.at[dst_v.at[b]],
                                 sem_sw.at[p], add=True),
            )
        for d in sc_descs[(SBB - 1) & 1]:
            d.wait()
        return c1

    lax.fori_loop(0, nsb, superbatch, 0)
    plsc.subcore_barrier()

    pltpu.sync_copy(out_sh.at[pl.ds(base, ORT)],
                    out_hbm.at[cid, pl.ds(base, ORT)])
    pltpu.sync_copy(den_sh.at[pl.ds(dbase, DPT)],
                    den_hbm.at[pl.ds(cid * NPAD + dbase, DPT)])


def _make_sc_gat():
    mesh = plsc.VectorSubcoreMesh(core_axis_name="c", subcore_axis_name="s")
    return pl.kernel(
        _sc_gat_body,
        out_type=(jax.ShapeDtypeStruct((NC, OROWS, H), _F32),
                  jax.ShapeDtypeStruct((NC * NPAD,), _F32)),
        mesh=mesh,
        compiler_params=pltpu.CompilerParams(needs_layout_passes=False),
        scratch_types=[
            pltpu.VMEM((SBB, K), jnp.int32),
            pltpu.VMEM((SBB, K), jnp.int32),
            pltpu.VMEM((2, K), _F32),
            pltpu.VMEM((2, K), _F32),
            pltpu.VMEM((2, K), _F32),
            pltpu.VMEM((2, K, H), _F32),
            pltpu.VMEM_SHARED((OROWS, H), _F32),
            pltpu.VMEM_SHARED((NPAD,), _F32),
            pltpu.VMEM_SHARED((NPAD,), _F32),
            pltpu.VMEM_SHARED((NPAD,), _F32),
            pltpu.SemaphoreType.DMA((2,)),
            pltpu.SemaphoreType.DMA((2,)),
            pltpu.SemaphoreType.DMA((2,)),
            pltpu.SemaphoreType.DMA((2,)),
            pltpu.SemaphoreType.DMA((2,)),
            pltpu.SemaphoreType.DMA((2,)),
            pltpu.SemaphoreType.DMA((2,)),
        ],
    )


def _tc1_body(x_ref, W_ref, as_ref, ad_ref, h_ref, asrc_ref, adst_ref):
    h = jnp.dot(x_ref[...], W_ref[...], preferred_element_type=_F32)
    h_ref[...] = h
    zpad = jnp.zeros((NPAD - N,), _F32)
    asrc_ref[...] = jnp.concatenate(
        [jnp.sum(h * as_ref[...][None, :], axis=1), zpad])
    adst_ref[...] = jnp.concatenate(
        [jnp.sum(h * ad_ref[...][None, :], axis=1), zpad])


def _tc2_body(op_ref, dp_ref, b_ref, W_ref, as_ref, ad_ref,
              h_ref, asrc_ref, adst_ref):
    osum = op_ref[0, :N, :] + op_ref[1, :N, :]
    den = dp_ref[0, :N] + dp_ref[1, :N] + 1e-16
    x2 = jnp.maximum(osum / den[:, None] + b_ref[...][None, :], 0.0)
    h = jnp.dot(x2, W_ref[...], preferred_element_type=_F32)
    h_ref[...] = h
    zpad = jnp.zeros((NPAD - N,), _F32)
    asrc_ref[...] = jnp.concatenate(
        [jnp.sum(h * as_ref[...][None, :], axis=1), zpad])
    adst_ref[...] = jnp.concatenate(
        [jnp.sum(h * ad_ref[...][None, :], axis=1), zpad])


def _tc3_body(op_ref, dp_ref, b_ref, mW1_ref, mb1_ref, mW2_ref, mb2_ref,
              batch_ref, lW1_ref, lb1_ref, lW2_ref, lb2_ref, out_ref):
    osum = op_ref[0, :N, :] + op_ref[1, :N, :]
    den = dp_ref[0, :N] + dp_ref[1, :N] + 1e-16
    x3 = jnp.maximum(osum / den[:, None] + b_ref[...][None, :], 0.0)
    t = jnp.maximum(
        jnp.dot(x3, mW1_ref[...], preferred_element_type=_F32)
        + mb1_ref[...][None, :], 0.0)
    node = (jnp.dot(t, mW2_ref[...], preferred_element_type=_F32)
            + mb2_ref[...][None, :])
    gids = lax.broadcasted_iota(jnp.int32, (G, N), 0)
    maskT = (gids == batch_ref[...][None, :]).astype(_F32)
    summed = jnp.dot(maskT, node, preferred_element_type=_F32)
    counts = jnp.sum(maskT, axis=1)
    pooled = summed / jnp.maximum(counts, 1.0)[:, None]
    t2 = jnp.maximum(
        jnp.dot(pooled, lW1_ref[...], preferred_element_type=_F32)
        + lb1_ref[...][None, :], 0.0)
    out_ref[...] = (jnp.dot(t2, lW2_ref[...], preferred_element_type=_F32)
                    + lb2_ref[...][None, :])


def kernel(x, edge_index, batch, W1, a_src1, a_dst1, b1, W2, a_src2, a_dst2,
           b2, mW1, mb1, mW2, mb2, lW1, lb1, lW2, lb2):
    ei = edge_index.astype(jnp.int32)
    batch32 = batch.astype(jnp.int32)
    loop = jnp.arange(N, dtype=jnp.int32)
    npad_e = EP - E - N
    src = jnp.concatenate(
        [ei[0], loop, jnp.zeros((npad_e,), jnp.int32)]).reshape(-1, SBB, K)
    dst = jnp.concatenate(
        [ei[1], loop, jnp.full((npad_e,), N, jnp.int32)]).reshape(-1, SBB, K)

    tc1 = pl.pallas_call(
        _tc1_body,
        out_shape=(jax.ShapeDtypeStruct((N, H), _F32),
                   jax.ShapeDtypeStruct((NPAD,), _F32),
                   jax.ShapeDtypeStruct((NPAD,), _F32)))
    tc2 = pl.pallas_call(
        _tc2_body,
        out_shape=(jax.ShapeDtypeStruct((N, H), _F32),
                   jax.ShapeDtypeStruct((NPAD,), _F32),
                   jax.ShapeDtypeStruct((NPAD,), _F32)))
    tc3 = pl.pallas_call(
        _tc3_body,
        out_shape=jax.ShapeDtypeStruct((G, C), _F32))
    sc_gat = _make_sc_gat()

    h1, asrc1, adst1 = tc1(x, W1, a_src1, a_dst1)
    op1, dp1 = sc_gat(h1, asrc1, adst1, src, dst)
    h2, asrc2, adst2 = tc2(op1, dp1.reshape(NC, NPAD), b1, W2,
                           a_src2, a_dst2)
    op2, dp2 = sc_gat(h2, asrc2, adst2, src, dst)
    logits = tc3(op2, dp2.reshape(NC, NPAD), b2, mW1, mb1, mW2, mb2, batch32,
                 lW1, lb1, lW2, lb2)
    return logits


# speedup vs baseline: 1.0670x; 1.0670x over previous
import functools

import jax
import jax.numpy as jnp
from jax import lax
from jax.experimental import pallas as pl
from jax.experimental.pallas import tpu as pltpu
from jax.experimental.pallas import tpu_sc as plsc

N = 10000
E = 320000
D = 128
H = 128
C = 10
G = 64

NC = 2
NS = 16
NW = NC * NS

K = 128
NB = 81
SBB = 9
EPT = NB * K
EP = NW * EPT
NSBT = NW * (NB // SBB)
FAST = 0
NSB_F = 11
NSB_S = 7
NPAD = 10240
OROWS = 10112
ORT = OROWS // NS
DPT = NPAD // NS

_F32 = jnp.float32


def _sc_gat_body(h_hbm, asrc_hbm, adst_hbm, src_hbm, dst_hbm,
                 out_hbm, den_hbm,
                 src_v, dst_v, asb_v, adb_v, w_v, rows_v,
                 out_sh, den_sh, asrc_sh, adst_sh,
                 sem_is, sem_id, sem_as, sem_ad, sem_g, sem_so, sem_sw):
    cid = lax.axis_index("c")
    tid = lax.axis_index("s")
    wid = tid * NC + cid
    ebase = wid * EPT
    base = tid * ORT
    dbase = tid * DPT

    z16 = jnp.zeros((16,), _F32)

    def zrow(r, c2):
        for c in range(8):
            rows_v[0, r, pl.ds(c * 16, 16)] = z16
        return c2

    lax.fori_loop(0, K, zrow, 0)
    for c in range(8):
        w_v[0, pl.ds(c * 16, 16)] = z16

    @pl.when(tid == 0)
    def _():
        pltpu.sync_copy(asrc_hbm, asrc_sh)
        pltpu.sync_copy(adst_hbm, adst_sh)

    nfull = ORT // K
    for jj in range(nfull):
        pltpu.sync_copy(rows_v.at[0], out_sh.at[pl.ds(base + jj * K, K)])
    rem = ORT - nfull * K
    if rem:
        pltpu.sync_copy(rows_v.at[0, pl.ds(0, rem)],
                        out_sh.at[pl.ds(base + nfull * K, rem)])
    for jj in range(DPT // K):
        pltpu.sync_copy(w_v.at[0], den_sh.at[pl.ds(dbase + jj * K, K)])
    plsc.subcore_barrier()

    def _issue_scalar_gathers(b, p):
        cp_as = pltpu.async_copy(asrc_sh.at[src_v.at[b]], asb_v.at[p],
                                 sem_as.at[p])
        cp_ad = pltpu.async_copy(adst_sh.at[dst_v.at[b]], adb_v.at[p],
                                 sem_ad.at[p])
        return cp_as, cp_ad

    is_fast = cid == FAST
    sb_base = jnp.where(is_fast, tid * NSB_F, NS * NSB_F + tid * NSB_S)
    nsb = jnp.where(is_fast, NSB_F, NSB_S)

    def superbatch(sb, c1):
        sb0 = sb_base + sb
        pltpu.sync_copy(src_hbm.at[sb0], src_v)
        pltpu.sync_copy(dst_hbm.at[sb0], dst_v)

        sc_descs = [None, None]
        g_descs = [None, None]
        a_descs = [None, None]
        a_descs[0] = _issue_scalar_gathers(0, 0)
        g_descs[0] = pltpu.async_copy(h_hbm.at[src_v.at[0]], rows_v.at[0],
                                      sem_g.at[0])
        for b in range(SBB):
            p = b & 1
            q = 1 - p
            if b >= 1:
                for d in sc_descs[q]:
                    d.wait()
            if b + 1 < SBB:
                a_descs[q] = _issue_scalar_gathers(b + 1, q)
            for d in a_descs[p]:
                d.wait()

            def wcomp(g, c3, _p=p):
                sl = pl.ds(g * 16, 16)
                e = asb_v[_p, sl] + adb_v[_p, sl]
                e = jnp.where(e > 0, e, 0.2 * e)
                w_v[_p, sl] = jnp.exp(e)
                return c3

            lax.fori_loop(0, K // 16, wcomp, 0)

            if b + 1 < SBB:
                g_descs[q] = pltpu.async_copy(h_hbm.at[src_v.at[b + 1]],
                                              rows_v.at[q], sem_g.at[q])
            g_descs[p].wait()

            def scale(g, c3, _p=p):
                wv16 = w_v[_p, pl.ds(g * 16, 16)]
                for k in range(16):
                    wsc = wv16[k]
                    r = g * 16 + k
                    for c in range(8):
                        slc = pl.ds(c * 16, 16)
                        rows_v[_p, r, slc] = rows_v[_p, r, slc] * wsc
                return c3

            lax.fori_loop(0, K // 16, scale, 0)
            sc_descs[p] = (
                pltpu.async_copy(rows_v.at[p], out_sh.at[dst_v.at[b]],
                                 sem_so.at[p], add=True),
                pltpu.async_copy(w_v.at[p], den_sh.at[dst_v.at[b]],
                                 sem_sw.at[p], add=True),
            )
        for d in sc_descs[(SBB - 1) & 1]:
            d.wait()
        return c1

    lax.fori_loop(0, nsb, superbatch, 0)
    plsc.subcore_barrier()

    pltpu.sync_copy(out_sh.at[pl.ds(base, ORT)],
                    out_hbm.at[cid, pl.ds(base, ORT)])
    pltpu.sync_copy(den_sh.at[pl.ds(dbase, DPT)],
                    den_hbm.at[pl.ds(cid * NPAD + dbase, DPT)])


def _make_sc_gat():
    mesh = plsc.VectorSubcoreMesh(core_axis_name="c", subcore_axis_name="s")
    return pl.kernel(
        _sc_gat_body,
        out_type=(jax.ShapeDtypeStruct((NC, OROWS, H), _F32),
                  jax.ShapeDtypeStruct((NC * NPAD,), _F32)),
        mesh=mesh,
        compiler_params=pltpu.CompilerParams(needs_layout_passes=False),
        scratch_types=[
            pltpu.VMEM((SBB, K), jnp.int32),
            pltpu.VMEM((SBB, K), jnp.int32),
            pltpu.VMEM((2, K), _F32),
            pltpu.VMEM((2, K), _F32),
            pltpu.VMEM((2, K), _F32),
            pltpu.VMEM((2, K, H), _F32),
            pltpu.VMEM_SHARED((OROWS, H), _F32),
            pltpu.VMEM_SHARED((NPAD,), _F32),
            pltpu.VMEM_SHARED((NPAD,), _F32),
            pltpu.VMEM_SHARED((NPAD,), _F32),
            pltpu.SemaphoreType.DMA((2,)),
            pltpu.SemaphoreType.DMA((2,)),
            pltpu.SemaphoreType.DMA((2,)),
            pltpu.SemaphoreType.DMA((2,)),
            pltpu.SemaphoreType.DMA((2,)),
            pltpu.SemaphoreType.DMA((2,)),
            pltpu.SemaphoreType.DMA((2,)),
        ],
    )


def _tc1_body(x_ref, W_ref, as_ref, ad_ref, h_ref, asrc_ref, adst_ref):
    h = jnp.dot(x_ref[...], W_ref[...], preferred_element_type=_F32)
    h_ref[...] = h
    zpad = jnp.zeros((NPAD - N,), _F32)
    asrc_ref[...] = jnp.concatenate(
        [jnp.sum(h * as_ref[...][None, :], axis=1), zpad])
    adst_ref[...] = jnp.concatenate(
        [jnp.sum(h * ad_ref[...][None, :], axis=1), zpad])


def _tc2_body(op_ref, dp_ref, b_ref, W_ref, as_ref, ad_ref,
              h_ref, asrc_ref, adst_ref):
    osum = op_ref[0, :N, :] + op_ref[1, :N, :]
    den = dp_ref[0, :N] + dp_ref[1, :N] + 1e-16
    x2 = jnp.maximum(osum / den[:, None] + b_ref[...][None, :], 0.0)
    h = jnp.dot(x2, W_ref[...], preferred_element_type=_F32)
    h_ref[...] = h
    zpad = jnp.zeros((NPAD - N,), _F32)
    asrc_ref[...] = jnp.concatenate(
        [jnp.sum(h * as_ref[...][None, :], axis=1), zpad])
    adst_ref[...] = jnp.concatenate(
        [jnp.sum(h * ad_ref[...][None, :], axis=1), zpad])


def _tc3_body(op_ref, dp_ref, b_ref, mW1_ref, mb1_ref, mW2_ref, mb2_ref,
              batch_ref, lW1_ref, lb1_ref, lW2_ref, lb2_ref, out_ref):
    osum = op_ref[0, :N, :] + op_ref[1, :N, :]
    den = dp_ref[0, :N] + dp_ref[1, :N] + 1e-16
    x3 = jnp.maximum(osum / den[:, None] + b_ref[...][None, :], 0.0)
    t = jnp.maximum(
        jnp.dot(x3, mW1_ref[...], preferred_element_type=_F32)
        + mb1_ref[...][None, :], 0.0)
    node = (jnp.dot(t, mW2_ref[...], preferred_element_type=_F32)
            + mb2_ref[...][None, :])
    gids = lax.broadcasted_iota(jnp.int32, (G, N), 0)
    maskT = (gids == batch_ref[...][None, :]).astype(_F32)
    summed = jnp.dot(maskT, node, preferred_element_type=_F32)
    counts = jnp.sum(maskT, axis=1)
    pooled = summed / jnp.maximum(counts, 1.0)[:, None]
    t2 = jnp.maximum(
        jnp.dot(pooled, lW1_ref[...], preferred_element_type=_F32)
        + lb1_ref[...][None, :], 0.0)
    out_ref[...] = (jnp.dot(t2, lW2_ref[...], preferred_element_type=_F32)
                    + lb2_ref[...][None, :])


def kernel(x, edge_index, batch, W1, a_src1, a_dst1, b1, W2, a_src2, a_dst2,
           b2, mW1, mb1, mW2, mb2, lW1, lb1, lW2, lb2):
    ei = edge_index.astype(jnp.int32)
    batch32 = batch.astype(jnp.int32)
    loop = jnp.arange(N, dtype=jnp.int32)
    npad_e = EP - E - N
    src = jnp.concatenate(
        [ei[0], loop, jnp.zeros((npad_e,), jnp.int32)]).reshape(-1, SBB, K)
    dst = jnp.concatenate(
        [ei[1], loop, jnp.full((npad_e,), N, jnp.int32)]).reshape(-1, SBB, K)

    tc1 = pl.pallas_call(
        _tc1_body,
        out_shape=(jax.ShapeDtypeStruct((N, H), _F32),
                   jax.ShapeDtypeStruct((NPAD,), _F32),
                   jax.ShapeDtypeStruct((NPAD,), _F32)))
    tc2 = pl.pallas_call(
        _tc2_body,
        out_shape=(jax.ShapeDtypeStruct((N, H), _F32),
                   jax.ShapeDtypeStruct((NPAD,), _F32),
                   jax.ShapeDtypeStruct((NPAD,), _F32)))
    tc3 = pl.pallas_call(
        _tc3_body,
        out_shape=jax.ShapeDtypeStruct((G, C), _F32))
    sc_gat = _make_sc_gat()

    h1, asrc1, adst1 = tc1(x, W1, a_src1, a_dst1)
    op1, dp1 = sc_gat(h1, asrc1, adst1, src, dst)
    h2, asrc2, adst2 = tc2(op1, dp1.reshape(NC, NPAD), b1, W2,
                           a_src2, a_dst2)
    op2, dp2 = sc_gat(h2, asrc2, adst2, src, dst)
    logits = tc3(op2, dp2.reshape(NC, NPAD), b2, mW1, mb1, mW2, mb2, batch32,
                 lW1, lb1, lW2, lb2)
    return logits
